# baseline (speedup 1.0000x reference)
import functools

import jax
import jax.numpy as jnp
from jax import lax
from jax.experimental import pallas as pl
from jax.experimental.pallas import tpu as pltpu
from jax.experimental.pallas import tpu_sc as plsc

_B = 1024
_F = 4 * 64 * 64
_R = 128
_G = _B // _R
_NW = 32
_CHUNK = _B // _NW
_TBL = 1024
_F32MAX = 3.4028234663852886e38


def _sc_gather_body(stats_hbm, t_hbm, out_hbm, table_v, idx_v, out_v):
    wid = lax.axis_index("s") * 2 + lax.axis_index("c")
    base = wid * _CHUNK
    pltpu.sync_copy(stats_hbm, table_v)
    pltpu.sync_copy(t_hbm.at[pl.ds(base, _CHUNK)], idx_v)
    for j in range(_CHUNK // 16):
        iv = idx_v[pl.ds(j * 16, 16)]
        out_v[pl.ds(j * 16, 16)] = plsc.load_gather(table_v, [iv])
    pltpu.sync_copy(out_v, out_hbm.at[pl.ds(base, _CHUNK)])


@jax.jit
def _sc_gather(stats_pad, t32):
    mesh = plsc.VectorSubcoreMesh(core_axis_name="c", subcore_axis_name="s")
    k = functools.partial(
        pl.kernel,
        mesh=mesh,
        out_type=jax.ShapeDtypeStruct((_B,), jnp.float32),
        scratch_types=[
            pltpu.VMEM((_TBL,), jnp.float32),
            pltpu.VMEM((_CHUNK,), jnp.int32),
            pltpu.VMEM((_CHUNK,), jnp.float32),
        ],
        compiler_params=pltpu.CompilerParams(needs_layout_passes=False),
    )(_sc_gather_body)
    return k(stats_pad, t32)


def _tc_body(stats_ref, g_ref, pred_ref, targ_ref,
             mse_ref, rmse_ref, tnorm_ref, pnorm_ref, r2_ref, loss_ref):
    i = pl.program_id(0)
    p = pred_ref[...]
    t = targ_ref[...]
    d = p - t
    inv = jnp.float32(1.0 / _F)
    mse = jnp.sum(d * d, axis=1) * inv
    pvar = jnp.sum(p * p, axis=1) * inv
    tvar = jnp.sum(t * t, axis=1) * inv
    mse_ref[0, 0, :] = mse
    rmse_ref[0, 0, :] = jnp.sqrt(mse)
    pnorm_ref[0, 0, :] = jnp.sqrt(pvar)
    tnorm_ref[0, 0, :] = jnp.sqrt(tvar)
    r2_ref[0, 0, :] = 1.0 - mse / tvar

    s = stats_ref[...]
    isn = s != s
    m = (jnp.sum(jnp.where(isn, 0.0, s))
         / jnp.sum(jnp.where(isn, jnp.float32(0.0), jnp.float32(1.0))))
    dist = m / g_ref[0, 0, :]
    dist = jnp.where(dist != dist, jnp.float32(1.0), dist)
    dist = jnp.where(dist == jnp.inf, jnp.float32(_F32MAX), dist)
    dist = jnp.where(dist == -jnp.inf, jnp.float32(-_F32MAX), dist)
    part = jnp.sum(mse * dist) * jnp.float32(1.0 / _B)

    @pl.when(i == 0)
    def _():
        loss_ref[...] = jnp.zeros((1, 1), jnp.float32)

    loss_ref[...] += jnp.reshape(part, (1, 1))


def _tc_stats(stats2d, g2d, pred2, targ2):
    row = jax.ShapeDtypeStruct((_G, 1, _R), jnp.float32)
    return pl.pallas_call(
        _tc_body,
        grid=(_G,),
        in_specs=[
            pl.BlockSpec((8, 128), lambda i: (0, 0)),
            pl.BlockSpec((1, 1, _R), lambda i: (i, 0, 0)),
            pl.BlockSpec((_R, _F), lambda i: (i, 0)),
            pl.BlockSpec((_R, _F), lambda i: (i, 0)),
        ],
        out_specs=[
            pl.BlockSpec((1, 1, _R), lambda i: (i, 0, 0)),
            pl.BlockSpec((1, 1, _R), lambda i: (i, 0, 0)),
            pl.BlockSpec((1, 1, _R), lambda i: (i, 0, 0)),
            pl.BlockSpec((1, 1, _R), lambda i: (i, 0, 0)),
            pl.BlockSpec((1, 1, _R), lambda i: (i, 0, 0)),
            pl.BlockSpec((1, 1), lambda i: (0, 0)),
        ],
        out_shape=[row, row, row, row, row,
                   jax.ShapeDtypeStruct((1, 1), jnp.float32)],
        compiler_params=pltpu.CompilerParams(
            dimension_semantics=("arbitrary",)),
    )(stats2d, g2d, pred2, targ2)


def kernel(pred, target, stats_mse, t):
    pred2 = jnp.reshape(pred, (_B, _F))
    targ2 = jnp.reshape(target, (_B, _F))
    stats_pad = jnp.concatenate(
        [stats_mse.astype(jnp.float32),
         jnp.full((_TBL - stats_mse.shape[0],), jnp.nan, jnp.float32)])
    t32 = jnp.asarray(t, jnp.int32)
    g = _sc_gather(stats_pad, t32)
    stats2d = jnp.reshape(stats_pad, (8, 128))
    g2d = jnp.reshape(g, (_G, 1, _R))
    mse, rmse, tnorm, pnorm, r2, loss = _tc_stats(stats2d, g2d, pred2, targ2)
    return (jnp.reshape(loss, ()),
            jnp.reshape(mse, (_B,)),
            jnp.reshape(rmse, (_B,)),
            jnp.reshape(tnorm, (_B,)),
            jnp.reshape(pnorm, (_B,)),
            jnp.reshape(r2, (_B,)))
